# baseline (speedup 1.0000x reference)
import functools

import jax
import jax.numpy as jnp
from jax import lax
from jax.experimental import pallas as pl
from jax.experimental.pallas import tpu as pltpu
from jax.experimental.pallas import tpu_sc as plsc

_NUM_TO_REPLACE = 64
_B, _N, _C = 32, 8192, 128
_ROWS = _B * _N
_CHUNK = 256
_NCHUNKS = _N // _CHUNK


def _zero_row_ids():
    perm = jax.random.permutation(jax.random.key(42), _N)
    i_to_zero = perm[:_NUM_TO_REPLACE].astype(jnp.int32)
    rows = jnp.arange(_B, dtype=jnp.int32)[:, None] * _N + i_to_zero[None, :]
    return rows.reshape(-1)


def _body(pts_hbm, idx_hbm, zeros_hbm, out_hbm,
          buf0, buf1, idx_v, zeros_v, rsem0, rsem1, wsem0, wsem1, zsem):
    nc = 2
    wid = lax.axis_index("s") * nc + lax.axis_index("c")
    base = wid * _N
    bufs = (buf0, buf1)
    rsems = (rsem0, rsem1)
    wsems = (wsem0, wsem1)

    def read(g):
        return pltpu.make_async_copy(
            pts_hbm.at[pl.ds(base + g * _CHUNK, _CHUNK), :],
            bufs[g % 2], rsems[g % 2])

    def write(g):
        return pltpu.make_async_copy(
            bufs[g % 2],
            out_hbm.at[pl.ds(base + g * _CHUNK, _CHUNK), :],
            wsems[g % 2])

    pltpu.sync_copy(idx_hbm.at[pl.ds(wid * _NUM_TO_REPLACE, _NUM_TO_REPLACE)],
                    idx_v)
    pltpu.sync_copy(zeros_hbm, zeros_v)

    read(0).start()
    for g in range(_NCHUNKS):
        if g + 1 < _NCHUNKS:
            if g >= 1:
                write(g - 1).wait()
            read(g + 1).start()
        read(g).wait()
        write(g).start()
    write(_NCHUNKS - 2).wait()
    write(_NCHUNKS - 1).wait()

    pltpu.async_copy(zeros_v, out_hbm.at[idx_v], zsem).wait()


def _make_kernel():
    mesh = plsc.VectorSubcoreMesh(core_axis_name="c", subcore_axis_name="s")
    return functools.partial(
        pl.kernel,
        out_type=jax.ShapeDtypeStruct((_ROWS, _C), jnp.float32),
        mesh=mesh,
        scratch_types=[
            pltpu.VMEM((_CHUNK, _C), jnp.float32),
            pltpu.VMEM((_CHUNK, _C), jnp.float32),
            pltpu.VMEM((_NUM_TO_REPLACE,), jnp.int32),
            pltpu.VMEM((_NUM_TO_REPLACE, _C), jnp.float32),
            pltpu.SemaphoreType.DMA,
            pltpu.SemaphoreType.DMA,
            pltpu.SemaphoreType.DMA,
            pltpu.SemaphoreType.DMA,
            pltpu.SemaphoreType.DMA,
        ],
    )(_body)


_sc_call = _make_kernel()


def kernel(pts):
    flat = pts.reshape(_ROWS, _C)
    idx = _zero_row_ids()
    zeros = jnp.zeros((_NUM_TO_REPLACE, _C), jnp.float32)
    out = _sc_call(flat, idx, zeros)
    return out.reshape(_B, _N, _C)
